# Initial kernel scaffold; baseline (speedup 1.0000x reference)
import jax
import jax.numpy as jnp
from jax.experimental import pallas as pl


def kernel(logits, labels):
    raise NotImplementedError("write your pallas kernel here")


# speedup vs baseline: 18.3485x; 18.3485x over previous
import functools

import jax
import jax.numpy as jnp
from jax import lax
from jax.experimental import pallas as pl
from jax.experimental.pallas import tpu as pltpu
from jax.experimental.pallas import tpu_sc as plsc

P = 2097152
NC, NS, L = 2, 16, 16
NW = NC * NS
B = 16384
LO, HI = -15.0, 17.0
INVW = B / (HI - LO)
NB = 4 * B
PER_W = P // NW
CH = 8192
SL = NB // NW

_mesh = lambda: plsc.VectorSubcoreMesh(core_axis_name="c", subcore_axis_name="s")


def _wid():
    return lax.axis_index("s") * NC + lax.axis_index("c")


@functools.partial(
    pl.kernel,
    out_type=jax.ShapeDtypeStruct((NW, NB), jnp.float32),
    mesh=_mesh(),
    scratch_types=[
        pltpu.VMEM((NB,), jnp.float32),
        pltpu.VMEM((CH,), jnp.float32),
        pltpu.VMEM((CH,), jnp.int32),
    ],
    compiler_params=pltpu.CompilerParams(needs_layout_passes=False),
)
def _hist_kernel(lg_hbm, lab_hbm, out_hbm, hist_v, lg_v, lab_v):
    wid = _wid()
    base = wid * PER_W

    zeros = jnp.zeros((L,), jnp.float32)

    def zbody(i, _):
        hist_v[pl.ds(i * L, L)] = zeros
        return 0

    lax.fori_loop(0, NB // L, zbody, 0)

    ones = jnp.full((L,), 1.0, jnp.float32)

    def chunk_body(c, _):
        off = base + c * CH
        pltpu.sync_copy(lg_hbm.at[pl.ds(off, CH)], lg_v)
        pltpu.sync_copy(lab_hbm.at[pl.ds(off, CH)], lab_v)

        def ibody(i, _):
            lg = lg_v[pl.ds(i * L, L)]
            t = lab_v[pl.ds(i * L, L)]
            tf = t.astype(jnp.float32)
            e = 1.0 - lg * (2.0 * tf - 1.0)
            f = jnp.where(e > 0.0, e + 1.0, jnp.exp(e))
            b = jnp.clip(((HI - e) * INVW).astype(jnp.int32), 0, B - 1)
            bb = b + t * B
            plsc.addupdate_scatter(hist_v, [bb], ones)
            plsc.addupdate_scatter(hist_v, [bb + 2 * B], f)
            return 0

        lax.fori_loop(0, CH // L, ibody, 0)
        return 0

    lax.fori_loop(0, PER_W // CH, chunk_body, 0)
    pltpu.sync_copy(hist_v, out_hbm.at[wid])


@functools.partial(
    pl.kernel,
    out_type=jax.ShapeDtypeStruct((NB,), jnp.float32),
    mesh=_mesh(),
    scratch_types=[
        pltpu.VMEM((SL,), jnp.float32),
        pltpu.VMEM((SL,), jnp.float32),
    ],
    compiler_params=pltpu.CompilerParams(needs_layout_passes=False),
)
def _reduce_kernel(h_hbm, g_hbm, acc_v, tmp_v):
    wid = _wid()
    j0 = wid * SL
    pltpu.sync_copy(h_hbm.at[0, pl.ds(j0, SL)], acc_v)
    for w in range(1, NW):
        pltpu.sync_copy(h_hbm.at[w, pl.ds(j0, SL)], tmp_v)

        def abody(i, _):
            acc_v[pl.ds(i * L, L)] = acc_v[pl.ds(i * L, L)] + tmp_v[pl.ds(i * L, L)]
            return 0

        lax.fori_loop(0, SL // L, abody, 0)
    pltpu.sync_copy(acc_v, g_hbm.at[pl.ds(j0, SL)])


@functools.partial(
    pl.kernel,
    out_type=jax.ShapeDtypeStruct((L,), jnp.float32),
    mesh=_mesh(),
    scratch_types=[
        pltpu.VMEM((NB,), jnp.float32),
        pltpu.VMEM((L,), jnp.float32),
    ],
    compiler_params=pltpu.CompilerParams(needs_layout_passes=False),
)
def _scan_kernel(g_hbm, out_hbm, g_v, res_v):
    wid = _wid()

    @pl.when(wid == 0)
    def _():
        pltpu.sync_copy(g_hbm, g_v)

        def gbody(i, acc):
            return acc + g_v[pl.ds(B + i * L, L)]

        gvec = lax.fori_loop(0, B // L, gbody, jnp.zeros((L,), jnp.float32))
        G = jnp.sum(gvec)

        def sbody(i, carry):
            ps, cs, acc = carry
            cn = g_v[pl.ds(i * L, L)]
            cp = g_v[pl.ds(B + i * L, L)]
            fn = g_v[pl.ds(2 * B + i * L, L)]
            fp = g_v[pl.ds(3 * B + i * L, L)]
            n = cn + cp
            s = fn + fp
            cum_n = plsc.cumsum(n)
            cum_t = plsc.cumsum(cp)
            pe = ps + cum_n
            ce = cs + cum_t
            psl = pe - n
            csl = ce - cp
            num = (G - csl) * n + cp * psl
            us = G + psl - csl
            ue = G + pe - ce
            dj = num / jnp.maximum(us * ue, 1.0)
            fbar = s / jnp.maximum(n, 1.0)
            contrib = jnp.where(n > 0.0, fbar * dj, 0.0)
            contrib = jnp.where((G == 0.0) & (psl == 0.0) & (n > 0.0), fbar, contrib)
            return ps + jnp.sum(n), cs + jnp.sum(cp), acc + contrib

        _, _, acc = lax.fori_loop(
            0, B // L, sbody, (jnp.float32(0.0), jnp.float32(0.0), jnp.zeros((L,), jnp.float32))
        )
        res_v[...] = jnp.full((L,), jnp.sum(acc))
        pltpu.sync_copy(res_v, out_hbm)


def kernel(logits, labels):
    lab32 = labels.reshape(-1).astype(jnp.int32)
    lg = logits.reshape(-1)
    h = _hist_kernel(lg, lab32)
    g = _reduce_kernel(h)
    out = _scan_kernel(g)
    return out[0]
